# Initial kernel scaffold; baseline (speedup 1.0000x reference)
import functools

import jax
import jax.numpy as jnp
from jax import lax
from jax.experimental import pallas as pl
from jax.experimental.pallas import tpu as pltpu
from jax.experimental.pallas import tpu_sc as plsc

_N_NODES = 100000
_N_EDGES = 6400000
_NC = 2
_NS = 16
_L = 16
_NW = _NC * _NS
_EPW = _N_EDGES // _NW
_CHUNK = 4000
_NCHUNK = _EPW // _CHUNK
_GROUPS = _CHUNK // _L

_mesh = plsc.VectorSubcoreMesh(core_axis_name="c", subcore_axis_name="s")


@functools.partial(
    pl.kernel,
    out_type=(
        jax.ShapeDtypeStruct((_N_EDGES,), jnp.float32),
        jax.ShapeDtypeStruct((_N_EDGES, 3), jnp.float32),
    ),
    mesh=_mesh,
    scratch_types=[
        pltpu.VMEM((_CHUNK,), jnp.int32),
        pltpu.VMEM((_CHUNK,), jnp.int32),
        pltpu.VMEM((_CHUNK, 3), jnp.float32),
        pltpu.VMEM((_CHUNK, 3), jnp.float32),
        pltpu.VMEM((_CHUNK, 3), jnp.float32),
        pltpu.VMEM((_CHUNK,), jnp.float32),
        pltpu.SemaphoreType.DMA,
    ],
)
def _distance_kernel(xyz, ei, dis_out, vec_out,
                     idx_s, idx_d, buf_s, buf_d, vec_l, dis_l, sem):
    wid = lax.axis_index("s") * _NC + lax.axis_index("c")
    tile_base = wid * _EPW

    @pl.loop(0, _NCHUNK)
    def _chunk(ci):
        base = tile_base + ci * _CHUNK
        pltpu.sync_copy(ei.at[0, pl.ds(base, _CHUNK)], idx_s)
        pltpu.sync_copy(ei.at[1, pl.ds(base, _CHUNK)], idx_d)
        cs = pltpu.async_copy(xyz.at[idx_s], buf_s, sem)
        cd = pltpu.async_copy(xyz.at[idx_d], buf_d, sem)
        cs.wait()
        cd.wait()

        @pl.loop(0, _GROUPS)
        def _group(g):
            e = g * _L + lax.iota(jnp.int32, _L)
            c0 = jnp.zeros((_L,), jnp.int32)
            c1 = jnp.ones((_L,), jnp.int32)
            c2 = jnp.full((_L,), 2, jnp.int32)
            dx = plsc.load_gather(buf_s, [e, c0]) - plsc.load_gather(buf_d, [e, c0])
            dy = plsc.load_gather(buf_s, [e, c1]) - plsc.load_gather(buf_d, [e, c1])
            dz = plsc.load_gather(buf_s, [e, c2]) - plsc.load_gather(buf_d, [e, c2])
            plsc.store_scatter(vec_l, [e, c0], dx)
            plsc.store_scatter(vec_l, [e, c1], dy)
            plsc.store_scatter(vec_l, [e, c2], dz)
            s = dx * dx + dy * dy + dz * dz
            i = lax.bitcast_convert_type(s, jnp.int32)
            y = lax.bitcast_convert_type(0x5F3759DF - (i >> 1), jnp.float32)
            y = y * (1.5 - 0.5 * s * y * y)
            y = y * (1.5 - 0.5 * s * y * y)
            y = y * (1.5 - 0.5 * s * y * y)
            dis_l[pl.ds(g * _L, _L)] = jnp.where(s > 0.0, s * y, 0.0)

        pltpu.sync_copy(dis_l, dis_out.at[pl.ds(base, _CHUNK)])
        pltpu.sync_copy(vec_l, vec_out.at[pl.ds(base, _CHUNK)])


def kernel(xyz, edge_index):
    return _distance_kernel(xyz, edge_index)


# speedup vs baseline: 13.5120x; 13.5120x over previous
import functools

import jax
import jax.numpy as jnp
from jax import lax
from jax.experimental import pallas as pl
from jax.experimental.pallas import tpu as pltpu
from jax.experimental.pallas import tpu_sc as plsc

_N_NODES = 100000
_N_EDGES = 6400000
_NC = 2
_NS = 16
_L = 16
_NW = _NC * _NS
_EPW = _N_EDGES // _NW
_CHUNK = 4000
_NCHUNK = _EPW // _CHUNK
_GROUPS = _CHUNK // _L

_mesh = plsc.VectorSubcoreMesh(core_axis_name="c", subcore_axis_name="s")


@functools.partial(
    pl.kernel,
    out_type=(
        jax.ShapeDtypeStruct((_N_EDGES,), jnp.float32),
        jax.ShapeDtypeStruct((_N_EDGES, 3), jnp.float32),
    ),
    mesh=_mesh,
    scratch_types=[
        pltpu.VMEM((_CHUNK,), jnp.int32),
        pltpu.VMEM((_CHUNK,), jnp.int32),
        pltpu.VMEM((_CHUNK, 8), jnp.float32),
        pltpu.VMEM((_CHUNK, 8), jnp.float32),
        pltpu.VMEM((_CHUNK, 3), jnp.float32),
        pltpu.VMEM((_CHUNK,), jnp.float32),
        pltpu.SemaphoreType.DMA,
    ],
    compiler_params=pltpu.CompilerParams(
        needs_layout_passes=False, use_tc_tiling_on_sc=False),
)
def _distance_kernel(xyz, src, dst, dis_out, vec_out,
                     idx_s, idx_d, buf_s, buf_d, vec_l, dis_l, sem):
    wid = lax.axis_index("s") * _NC + lax.axis_index("c")
    tile_base = wid * _EPW

    @pl.loop(0, _NCHUNK)
    def _chunk(ci):
        base = tile_base + ci * _CHUNK
        pltpu.sync_copy(src.at[pl.ds(base, _CHUNK)], idx_s)
        pltpu.sync_copy(dst.at[pl.ds(base, _CHUNK)], idx_d)
        cs = pltpu.async_copy(xyz.at[idx_s], buf_s, sem)
        cd = pltpu.async_copy(xyz.at[idx_d], buf_d, sem)
        cs.wait()
        cd.wait()
        @pl.loop(0, _GROUPS)
        def _group(g):
            e = g * _L + lax.iota(jnp.int32, _L)
            c0 = jnp.zeros((_L,), jnp.int32)
            c1 = jnp.ones((_L,), jnp.int32)
            c2 = jnp.full((_L,), 2, jnp.int32)
            dx = plsc.load_gather(buf_s, [e, c0]) - plsc.load_gather(buf_d, [e, c0])
            dy = plsc.load_gather(buf_s, [e, c1]) - plsc.load_gather(buf_d, [e, c1])
            dz = plsc.load_gather(buf_s, [e, c2]) - plsc.load_gather(buf_d, [e, c2])
            plsc.store_scatter(vec_l, [e, c0], dx)
            plsc.store_scatter(vec_l, [e, c1], dy)
            plsc.store_scatter(vec_l, [e, c2], dz)
            s = dx * dx + dy * dy + dz * dz
            i = lax.bitcast_convert_type(s, jnp.int32)
            y = lax.bitcast_convert_type(0x5F3759DF - (i >> 1), jnp.float32)
            y = y * (1.5 - 0.5 * s * y * y)
            y = y * (1.5 - 0.5 * s * y * y)
            y = y * (1.5 - 0.5 * s * y * y)
            dis_l[pl.ds(g * _L, _L)] = jnp.where(s > 0.0, s * y, 0.0)

        pltpu.sync_copy(dis_l, dis_out.at[pl.ds(base, _CHUNK)])
        pltpu.sync_copy(vec_l, vec_out.at[pl.ds(base, _CHUNK)])


def kernel(xyz, edge_index):
    xyz8 = jnp.concatenate(
        [xyz, jnp.zeros((xyz.shape[0], 5), jnp.float32)], axis=1)
    return _distance_kernel(xyz8, edge_index[0], edge_index[1])
